# baseline (speedup 1.0000x reference)
import jax
import jax.numpy as jnp
from jax import lax
from jax.experimental import pallas as pl
from jax.experimental.pallas import tpu as pltpu
from jax.experimental.pallas import tpu_sc as plsc

N = 10000
E = 320000
D = 128
DE = 16
H = 64
EPS = 1e-7

NC = 2
NS = 16
NW = NC * NS
K = 2
EH = E // K
GC = 40
GPT = EH // NW
GCH = GPT // GC
SC_C = 80
CH = SC_C // 2
EPT = EH // NS
SCH = EPT // SC_C
ZT = 10
RPT = N // ZT


def _sc_gather_body(xh, srch, xsrc, ib0, ib1, xb0, xb1, isem, gsem, wsem):
    c = lax.axis_index("c")
    s = lax.axis_index("s")
    wid = s * NC + c
    base = wid * GPT

    ibufs = (ib0, ib1)
    xbufs = (xb0, xb1)

    def issue_idx(jj, b):
        pltpu.async_copy(srch.at[pl.ds(base + jj * GC, GC)], ibufs[b], isem)

    def wait_idx(jj, b):
        pltpu.make_async_copy(
            srch.at[pl.ds(base + jj * GC, GC)], ibufs[b], isem).wait()

    def issue_gather(b):
        pltpu.async_copy(xh.at[ibufs[b]], xbufs[b], gsem)

    def wait_gather(b):
        pltpu.make_async_copy(xh.at[ibufs[b]], xbufs[b], gsem).wait()

    def issue_write(jj, b):
        pltpu.async_copy(xbufs[b], xsrc.at[pl.ds(base + jj * GC, GC)], wsem)

    def wait_write(jj, b):
        pltpu.make_async_copy(
            xbufs[b], xsrc.at[pl.ds(base + jj * GC, GC)], wsem).wait()

    issue_idx(0, 0)
    issue_idx(1, 1)
    wait_idx(0, 0)
    issue_gather(0)

    @pl.loop(0, GCH + 1, step=2)
    def _chunk(j):
        for b in range(2):
            o = 1 - b
            jj = j + b

            @pl.when(jj < GCH)
            def _():
                wait_gather(b)

                @pl.when(jj + 1 < GCH)
                def _():
                    wait_idx(jj + 1, o)
                    issue_gather(o)

                @pl.when(jj + 2 < GCH)
                def _():
                    issue_idx(jj + 2, b)

                @pl.when(jj >= 2)
                def _():
                    wait_write(jj - 2, b)

                issue_write(jj, b)

    wait_write(GCH - 2, (GCH - 2) % 2)
    wait_write(GCH - 1, (GCH - 1) % 2)


def _sc_gather(x, src):
    mesh = plsc.VectorSubcoreMesh(core_axis_name="c", subcore_axis_name="s")
    f = pl.kernel(
        _sc_gather_body,
        out_type=jax.ShapeDtypeStruct((EH, D), jnp.float32),
        mesh=mesh,
        scratch_types=[
            pltpu.VMEM((GC,), jnp.int32),
            pltpu.VMEM((GC,), jnp.int32),
            pltpu.VMEM((GC, D), jnp.float32),
            pltpu.VMEM((GC, D), jnp.float32),
            pltpu.SemaphoreType.DMA,
            pltpu.SemaphoreType.DMA,
            pltpu.SemaphoreType.DMA,
        ],
    )
    return f(x, src)


_BE = 4000


def _edge_w_body(ea_ref, w_ref, xs_ref, t_ref, o_ref):
    e = jnp.dot(ea_ref[...], w_ref[...], preferred_element_type=jnp.float32)
    m = jnp.maximum(xs_ref[...] + e, 0.0) + EPS
    w = jnp.exp(t_ref[0, 0] * m)
    mw = m * w
    o_ref[0] = jnp.concatenate([w[:, :H], mw[:, :H]], axis=1)
    o_ref[1] = jnp.concatenate([w[:, H:], mw[:, H:]], axis=1)


def _edge_w(edge_attr, W_edge, xsrc, t11):
    return pl.pallas_call(
        _edge_w_body,
        grid=(EH // _BE,),
        in_specs=[
            pl.BlockSpec((_BE, DE), lambda i: (i, 0)),
            pl.BlockSpec((DE, D), lambda i: (0, 0)),
            pl.BlockSpec((_BE, D), lambda i: (i, 0)),
            pl.BlockSpec((1, 1), lambda i: (0, 0)),
        ],
        out_specs=pl.BlockSpec((2, _BE, D), lambda i: (0, i, 0)),
        out_shape=jax.ShapeDtypeStruct((2, EH, D), jnp.float32),
    )(edge_attr, W_edge, xsrc, t11)


def _sc_scatter_body(wmwh0, wmwh1, dsth0, dsth1, wm, acc,
                     db00, db01, db10, db11,
                     wb00, wb01, wb10, wb11, zb,
                     dsem, vsem, ssem):
    c = lax.axis_index("c")
    s = lax.axis_index("s")

    zero = jnp.zeros((16,), jnp.float32)
    for r in range(CH):
        for k in range(D // 16):
            zb[r, pl.ds(k * 16, 16)] = zero

    @pl.when(s < ZT)
    def _zero():
        @pl.loop(0, RPT // CH)
        def _(q):
            pltpu.sync_copy(zb, acc.at[pl.ds(s * RPT + q * CH, CH)])

    plsc.subcore_barrier()

    base0 = s * EPT
    dbufs = ((db00, db01), (db10, db11))
    wbufs = ((wb00, wb01), (wb10, wb11))

    def scatter_pass(wmwh, dsth):
        def issue_dst(jj, b):
            for h in range(2):
                pltpu.async_copy(
                    dsth.at[pl.ds(base0 + jj * SC_C + h * CH, CH)],
                    dbufs[b][h], dsem)

        def wait_dst(jj, b, h):
            pltpu.make_async_copy(
                dsth.at[pl.ds(base0 + jj * SC_C + h * CH, CH)], dbufs[b][h],
                dsem).wait()

        def issue_data(jj, b):
            for h in range(2):
                pltpu.async_copy(
                    wmwh.at[c, pl.ds(base0 + jj * SC_C + h * CH, CH)],
                    wbufs[b][h], vsem)

        def wait_data(jj, b, h):
            pltpu.make_async_copy(
                wmwh.at[c, pl.ds(base0 + jj * SC_C + h * CH, CH)],
                wbufs[b][h], vsem).wait()

        def issue_scatter(b, h):
            pltpu.async_copy(wbufs[b][h], acc.at[dbufs[b][h]], ssem, add=True)

        def wait_scatter(b, h):
            pltpu.make_async_copy(
                wbufs[b][h], acc.at[dbufs[b][h]], ssem).wait()

        issue_dst(0, 0)
        issue_dst(1, 1)
        issue_data(0, 0)
        issue_data(1, 1)

        @pl.loop(0, SCH + 1, step=2)
        def _chunk(j):
            for b in range(2):
                jj = j + b

                @pl.when(jj < SCH)
                def _():
                    for h in range(2):
                        wait_data(jj, b, h)
                        wait_dst(jj, b, h)
                        issue_scatter(b, h)

                    @pl.when(jj + 2 < SCH)
                    def _():
                        wait_scatter(b, 0)
                        wait_scatter(b, 1)
                        issue_dst(jj + 2, b)
                        issue_data(jj + 2, b)

        for b in range(2):
            wait_scatter(b, 0)
            wait_scatter(b, 1)

    scatter_pass(wmwh0, dsth0)
    scatter_pass(wmwh1, dsth1)

    plsc.subcore_barrier()

    @pl.when(s < ZT)
    def _copy_out():
        @pl.loop(0, RPT // CH)
        def _out(q):
            row = s * RPT + q * CH
            pltpu.sync_copy(acc.at[pl.ds(row, CH)], wm.at[c, pl.ds(row, CH)])


def _sc_scatter(wmw0, wmw1, dst0, dst1):
    mesh = plsc.VectorSubcoreMesh(core_axis_name="c", subcore_axis_name="s")
    f = pl.kernel(
        _sc_scatter_body,
        out_type=jax.ShapeDtypeStruct((2, N, D), jnp.float32),
        mesh=mesh,
        scratch_types=[
            pltpu.VMEM_SHARED((N, D), jnp.float32),
            pltpu.VMEM((CH,), jnp.int32),
            pltpu.VMEM((CH,), jnp.int32),
            pltpu.VMEM((CH,), jnp.int32),
            pltpu.VMEM((CH,), jnp.int32),
            pltpu.VMEM((CH, D), jnp.float32),
            pltpu.VMEM((CH, D), jnp.float32),
            pltpu.VMEM((CH, D), jnp.float32),
            pltpu.VMEM((CH, D), jnp.float32),
            pltpu.VMEM((CH, D), jnp.float32),
            pltpu.SemaphoreType.DMA,
            pltpu.SemaphoreType.DMA,
            pltpu.SemaphoreType.DMA,
        ],
    )
    return f(wmw0, wmw1, dst0, dst1)


_BN = 1000


def _stage_a_body(wm0_ref, wm1_ref, x_ref, w1_ref, out_ref, stats_ref):
    i = pl.program_id(0)
    wm0 = wm0_ref[0]
    wm1 = wm1_ref[0]
    den = jnp.concatenate([wm0[:, :H], wm1[:, :H]], axis=1)
    num = jnp.concatenate([wm0[:, H:], wm1[:, H:]], axis=1)
    out = num / (den + 1e-16) + x_ref[...]
    out_ref[...] = out
    h = jnp.dot(out, w1_ref[...], preferred_element_type=jnp.float32)
    st = jnp.concatenate(
        [jnp.sum(h, axis=0, keepdims=True),
         jnp.sum(h * h, axis=0, keepdims=True)], axis=0)

    @pl.when(i == 0)
    def _():
        stats_ref[...] = st

    @pl.when(i > 0)
    def _():
        stats_ref[...] += st


def _stage_a(wm, x, W1):
    return pl.pallas_call(
        _stage_a_body,
        grid=(N // _BN,),
        in_specs=[
            pl.BlockSpec((1, _BN, D), lambda i: (0, i, 0)),
            pl.BlockSpec((1, _BN, D), lambda i: (1, i, 0)),
            pl.BlockSpec((_BN, D), lambda i: (i, 0)),
            pl.BlockSpec((D, 2 * D), lambda i: (0, 0)),
        ],
        out_specs=[
            pl.BlockSpec((_BN, D), lambda i: (i, 0)),
            pl.BlockSpec((2, 2 * D), lambda i: (0, 0)),
        ],
        out_shape=[
            jax.ShapeDtypeStruct((N, D), jnp.float32),
            jax.ShapeDtypeStruct((2, 2 * D), jnp.float32),
        ],
    )(wm, wm, x, W1)


def _stage_b_body(out_ref, st_ref, w1_ref, g1_ref, b1_ref, w2_ref, gln_ref,
                  bln_ref, y_ref):
    h = jnp.dot(out_ref[...], w1_ref[...], preferred_element_type=jnp.float32)
    s = st_ref[...]
    mu = s[0:1, :] / N
    var = s[1:2, :] / N - mu * mu
    h = (h - mu) * lax.rsqrt(var + 1e-5) * g1_ref[...] + b1_ref[...]
    h = jnp.maximum(h, 0.0)
    h2 = jnp.dot(h, w2_ref[...], preferred_element_type=jnp.float32)
    mu2 = jnp.mean(h2, axis=1, keepdims=True)
    var2 = jnp.mean((h2 - mu2) ** 2, axis=1, keepdims=True)
    hn = (h2 - mu2) * lax.rsqrt(var2 + 1e-5) * gln_ref[...] + bln_ref[...]
    y_ref[...] = jnp.where(hn > 0.0, hn, jnp.exp(jnp.minimum(hn, 0.0)) - 1.0)


def _stage_b(out, stats, W1, gamma1, beta1, W2, gamma_ln, beta_ln):
    return pl.pallas_call(
        _stage_b_body,
        grid=(N // _BN,),
        in_specs=[
            pl.BlockSpec((_BN, D), lambda i: (i, 0)),
            pl.BlockSpec((2, 2 * D), lambda i: (0, 0)),
            pl.BlockSpec((D, 2 * D), lambda i: (0, 0)),
            pl.BlockSpec((2 * D,), lambda i: (0,)),
            pl.BlockSpec((2 * D,), lambda i: (0,)),
            pl.BlockSpec((2 * D, D), lambda i: (0, 0)),
            pl.BlockSpec((D,), lambda i: (0,)),
            pl.BlockSpec((D,), lambda i: (0,)),
        ],
        out_specs=pl.BlockSpec((_BN, D), lambda i: (i, 0)),
        out_shape=jax.ShapeDtypeStruct((N, D), jnp.float32),
    )(out, stats, W1, gamma1, beta1, W2, gamma_ln, beta_ln)


def kernel(x, edge_index, edge_attr, W_edge, t, W1, gamma1, beta1, W2,
           gamma_ln, beta_ln):
    t11 = jnp.reshape(t, (1, 1)).astype(jnp.float32)
    src = edge_index[0]
    dst = edge_index[1]
    xsrc0 = _sc_gather(x, src[:EH])
    xsrc1 = _sc_gather(x, src[EH:])
    wmw0 = _edge_w(edge_attr[:EH], W_edge, xsrc0, t11)
    wmw1 = _edge_w(edge_attr[EH:], W_edge, xsrc1, t11)
    wm = _sc_scatter(wmw0, wmw1, dst[:EH], dst[EH:])
    out, stats = _stage_a(wm, x, W1)
    return _stage_b(out, stats, W1, gamma1, beta1, W2, gamma_ln, beta_ln)


# speedup vs baseline: 1.1772x; 1.1772x over previous
import jax
import jax.numpy as jnp
from jax import lax
from jax.experimental import pallas as pl
from jax.experimental.pallas import tpu as pltpu
from jax.experimental.pallas import tpu_sc as plsc

N = 10000
E = 320000
D = 128
DE = 16
H = 64
EPS = 1e-7

NC = 2
NS = 16
NW = NC * NS
GPT = E // NW
C = 80
GCH = GPT // C
EPT = E // NS
SCH = EPT // C
CH = C // 2
ZT = 10
RPT = N // ZT


def _sc_gather_body(xh, srch, xsrc, ib0, ib1, ib2, xb0, xb1, xb2,
                    isem, gsem, wsem):
    c = lax.axis_index("c")
    s = lax.axis_index("s")
    wid = s * NC + c
    base = wid * GPT

    ibufs = (ib0, ib1, ib2)
    xbufs = (xb0, xb1, xb2)

    def issue_idx(jj, b):
        pltpu.async_copy(srch.at[pl.ds(base + jj * C, C)], ibufs[b], isem)

    def wait_idx(jj, b):
        pltpu.make_async_copy(
            srch.at[pl.ds(base + jj * C, C)], ibufs[b], isem).wait()

    def issue_gather(b):
        pltpu.async_copy(xh.at[ibufs[b]], xbufs[b], gsem)

    def wait_gather(b):
        pltpu.make_async_copy(xh.at[ibufs[b]], xbufs[b], gsem).wait()

    def issue_write(jj, b):
        pltpu.async_copy(xbufs[b], xsrc.at[pl.ds(base + jj * C, C)], wsem)

    def wait_write(jj, b):
        pltpu.make_async_copy(
            xbufs[b], xsrc.at[pl.ds(base + jj * C, C)], wsem).wait()

    issue_idx(0, 0)
    issue_idx(1, 1)
    issue_idx(2, 2)
    wait_idx(0, 0)
    issue_gather(0)
    wait_idx(1, 1)
    issue_gather(1)

    @pl.loop(0, GCH + 1, step=3)
    def _chunk(j):
        for b0 in range(3):
            jj = j + b0
            b = b0

            @pl.when(jj < GCH)
            def _():
                wait_gather(b)
                issue_write(jj, b)

                @pl.when(jj + 3 < GCH)
                def _():
                    issue_idx(jj + 3, b)

                @pl.when(jj + 2 < GCH)
                def _():
                    nb = (b0 + 2) % 3

                    @pl.when(jj >= 1)
                    def _():
                        wait_write(jj - 1, (b0 + 2) % 3)

                    wait_idx(jj + 2, nb)
                    issue_gather(nb)

    wait_write(GCH - 3, (GCH - 3) % 3)
    wait_write(GCH - 2, (GCH - 2) % 3)
    wait_write(GCH - 1, (GCH - 1) % 3)


def _sc_gather(x, src):
    mesh = plsc.VectorSubcoreMesh(core_axis_name="c", subcore_axis_name="s")
    f = pl.kernel(
        _sc_gather_body,
        out_type=jax.ShapeDtypeStruct((E, D), jnp.float32),
        mesh=mesh,
        scratch_types=[
            pltpu.VMEM((C,), jnp.int32),
            pltpu.VMEM((C,), jnp.int32),
            pltpu.VMEM((C,), jnp.int32),
            pltpu.VMEM((C, D), jnp.float32),
            pltpu.VMEM((C, D), jnp.float32),
            pltpu.VMEM((C, D), jnp.float32),
            pltpu.SemaphoreType.DMA,
            pltpu.SemaphoreType.DMA,
            pltpu.SemaphoreType.DMA,
        ],
    )
    return f(x, src)


_BE = 4000


def _edge_w_body(ea_ref, w_ref, xs_ref, t_ref, o_ref):
    e = jnp.dot(ea_ref[...], w_ref[...], preferred_element_type=jnp.float32)
    m = jnp.maximum(xs_ref[...] + e, 0.0) + EPS
    w = jnp.exp(t_ref[0, 0] * m)
    mw = m * w
    o_ref[0] = jnp.concatenate([w[:, :H], mw[:, :H]], axis=1)
    o_ref[1] = jnp.concatenate([w[:, H:], mw[:, H:]], axis=1)


def _edge_w(edge_attr, W_edge, xsrc, t11):
    return pl.pallas_call(
        _edge_w_body,
        grid=(E // _BE,),
        in_specs=[
            pl.BlockSpec((_BE, DE), lambda i: (i, 0)),
            pl.BlockSpec((DE, D), lambda i: (0, 0)),
            pl.BlockSpec((_BE, D), lambda i: (i, 0)),
            pl.BlockSpec((1, 1), lambda i: (0, 0)),
        ],
        out_specs=pl.BlockSpec((2, _BE, D), lambda i: (0, i, 0)),
        out_shape=jax.ShapeDtypeStruct((2, E, D), jnp.float32),
    )(edge_attr, W_edge, xsrc, t11)


def _sc_scatter_body(wmwh, dsth, wm, acc,
                     db00, db01, db10, db11, db20, db21, db30, db31,
                     wb00, wb01, wb10, wb11, wb20, wb21, wb30, wb31, zb,
                     dsem, vsem, ssem):
    c = lax.axis_index("c")
    s = lax.axis_index("s")

    zero = jnp.zeros((16,), jnp.float32)
    for r in range(CH):
        for k in range(D // 16):
            zb[r, pl.ds(k * 16, 16)] = zero

    @pl.when(s < ZT)
    def _zero():
        @pl.loop(0, RPT // CH)
        def _(q):
            pltpu.sync_copy(zb, acc.at[pl.ds(s * RPT + q * CH, CH)])

    plsc.subcore_barrier()

    base0 = s * EPT
    dbufs = ((db00, db01), (db10, db11), (db20, db21), (db30, db31))
    wbufs = ((wb00, wb01), (wb10, wb11), (wb20, wb21), (wb30, wb31))

    def issue_dst(jj, b):
        for h in range(2):
            pltpu.async_copy(
                dsth.at[pl.ds(base0 + jj * C + h * CH, CH)], dbufs[b][h], dsem)

    def wait_dst(jj, b, h):
        pltpu.make_async_copy(
            dsth.at[pl.ds(base0 + jj * C + h * CH, CH)], dbufs[b][h],
            dsem).wait()

    def issue_data(jj, b):
        for h in range(2):
            pltpu.async_copy(
                wmwh.at[c, pl.ds(base0 + jj * C + h * CH, CH)], wbufs[b][h],
                vsem)

    def wait_data(jj, b, h):
        pltpu.make_async_copy(
            wmwh.at[c, pl.ds(base0 + jj * C + h * CH, CH)], wbufs[b][h],
            vsem).wait()

    def issue_scatter(b, h):
        pltpu.async_copy(wbufs[b][h], acc.at[dbufs[b][h]], ssem, add=True)

    def wait_scatter(b, h):
        pltpu.make_async_copy(wbufs[b][h], acc.at[dbufs[b][h]], ssem).wait()

    for b in range(4):
        issue_dst(b, b)
        issue_data(b, b)

    @pl.loop(0, SCH + 2, step=4)
    def _chunk(j):
        for b in range(4):
            jj = j + b

            @pl.when(jj < SCH)
            def _():
                for h in range(2):
                    wait_data(jj, b, h)
                    wait_dst(jj, b, h)
                    issue_scatter(b, h)

                @pl.when(jj + 4 < SCH)
                def _():
                    wait_scatter(b, 0)
                    wait_scatter(b, 1)
                    issue_dst(jj + 4, b)
                    issue_data(jj + 4, b)

    for b in range(4):
        wait_scatter(b, 0)
        wait_scatter(b, 1)

    plsc.subcore_barrier()

    @pl.when(s < ZT)
    def _copy_out():
        @pl.loop(0, RPT // CH)
        def _out(q):
            row = s * RPT + q * CH
            pltpu.sync_copy(acc.at[pl.ds(row, CH)], wm.at[c, pl.ds(row, CH)])


def _sc_scatter(wmw, dst):
    mesh = plsc.VectorSubcoreMesh(core_axis_name="c", subcore_axis_name="s")
    f = pl.kernel(
        _sc_scatter_body,
        out_type=jax.ShapeDtypeStruct((2, N, D), jnp.float32),
        mesh=mesh,
        scratch_types=[
            pltpu.VMEM_SHARED((N, D), jnp.float32),
            pltpu.VMEM((CH,), jnp.int32),
            pltpu.VMEM((CH,), jnp.int32),
            pltpu.VMEM((CH,), jnp.int32),
            pltpu.VMEM((CH,), jnp.int32),
            pltpu.VMEM((CH,), jnp.int32),
            pltpu.VMEM((CH,), jnp.int32),
            pltpu.VMEM((CH,), jnp.int32),
            pltpu.VMEM((CH,), jnp.int32),
            pltpu.VMEM((CH, D), jnp.float32),
            pltpu.VMEM((CH, D), jnp.float32),
            pltpu.VMEM((CH, D), jnp.float32),
            pltpu.VMEM((CH, D), jnp.float32),
            pltpu.VMEM((CH, D), jnp.float32),
            pltpu.VMEM((CH, D), jnp.float32),
            pltpu.VMEM((CH, D), jnp.float32),
            pltpu.VMEM((CH, D), jnp.float32),
            pltpu.VMEM((CH, D), jnp.float32),
            pltpu.SemaphoreType.DMA,
            pltpu.SemaphoreType.DMA,
            pltpu.SemaphoreType.DMA,
        ],
    )
    return f(wmw, dst)


_BN = 1000


def _stage_a_body(wm0_ref, wm1_ref, x_ref, w1_ref, out_ref, stats_ref):
    i = pl.program_id(0)
    wm0 = wm0_ref[0]
    wm1 = wm1_ref[0]
    den = jnp.concatenate([wm0[:, :H], wm1[:, :H]], axis=1)
    num = jnp.concatenate([wm0[:, H:], wm1[:, H:]], axis=1)
    out = num / (den + 1e-16) + x_ref[...]
    out_ref[...] = out
    h = jnp.dot(out, w1_ref[...], preferred_element_type=jnp.float32)
    st = jnp.concatenate(
        [jnp.sum(h, axis=0, keepdims=True),
         jnp.sum(h * h, axis=0, keepdims=True)], axis=0)

    @pl.when(i == 0)
    def _():
        stats_ref[...] = st

    @pl.when(i > 0)
    def _():
        stats_ref[...] += st


def _stage_a(wm, x, W1):
    return pl.pallas_call(
        _stage_a_body,
        grid=(N // _BN,),
        in_specs=[
            pl.BlockSpec((1, _BN, D), lambda i: (0, i, 0)),
            pl.BlockSpec((1, _BN, D), lambda i: (1, i, 0)),
            pl.BlockSpec((_BN, D), lambda i: (i, 0)),
            pl.BlockSpec((D, 2 * D), lambda i: (0, 0)),
        ],
        out_specs=[
            pl.BlockSpec((_BN, D), lambda i: (i, 0)),
            pl.BlockSpec((2, 2 * D), lambda i: (0, 0)),
        ],
        out_shape=[
            jax.ShapeDtypeStruct((N, D), jnp.float32),
            jax.ShapeDtypeStruct((2, 2 * D), jnp.float32),
        ],
    )(wm, wm, x, W1)


def _stage_b_body(out_ref, st_ref, w1_ref, g1_ref, b1_ref, w2_ref, gln_ref,
                  bln_ref, y_ref):
    h = jnp.dot(out_ref[...], w1_ref[...], preferred_element_type=jnp.float32)
    s = st_ref[...]
    mu = s[0:1, :] / N
    var = s[1:2, :] / N - mu * mu
    h = (h - mu) * lax.rsqrt(var + 1e-5) * g1_ref[...] + b1_ref[...]
    h = jnp.maximum(h, 0.0)
    h2 = jnp.dot(h, w2_ref[...], preferred_element_type=jnp.float32)
    mu2 = jnp.mean(h2, axis=1, keepdims=True)
    var2 = jnp.mean((h2 - mu2) ** 2, axis=1, keepdims=True)
    hn = (h2 - mu2) * lax.rsqrt(var2 + 1e-5) * gln_ref[...] + bln_ref[...]
    y_ref[...] = jnp.where(hn > 0.0, hn, jnp.exp(jnp.minimum(hn, 0.0)) - 1.0)


def _stage_b(out, stats, W1, gamma1, beta1, W2, gamma_ln, beta_ln):
    return pl.pallas_call(
        _stage_b_body,
        grid=(N // _BN,),
        in_specs=[
            pl.BlockSpec((_BN, D), lambda i: (i, 0)),
            pl.BlockSpec((2, 2 * D), lambda i: (0, 0)),
            pl.BlockSpec((D, 2 * D), lambda i: (0, 0)),
            pl.BlockSpec((2 * D,), lambda i: (0,)),
            pl.BlockSpec((2 * D,), lambda i: (0,)),
            pl.BlockSpec((2 * D, D), lambda i: (0, 0)),
            pl.BlockSpec((D,), lambda i: (0,)),
            pl.BlockSpec((D,), lambda i: (0,)),
        ],
        out_specs=pl.BlockSpec((_BN, D), lambda i: (i, 0)),
        out_shape=jax.ShapeDtypeStruct((N, D), jnp.float32),
    )(out, stats, W1, gamma1, beta1, W2, gamma_ln, beta_ln)


def kernel(x, edge_index, edge_attr, W_edge, t, W1, gamma1, beta1, W2,
           gamma_ln, beta_ln):
    t11 = jnp.reshape(t, (1, 1)).astype(jnp.float32)
    xsrc = _sc_gather(x, edge_index[0])
    wmw = _edge_w(edge_attr, W_edge, xsrc, t11)
    wm = _sc_scatter(wmw, edge_index[1])
    out, stats = _stage_a(wm, x, W1)
    return _stage_b(out, stats, W1, gamma1, beta1, W2, gamma_ln, beta_ln)
